# baseline (speedup 1.0000x reference)
import functools

import jax
import jax.numpy as jnp
import numpy as np
from jax import lax
from jax.experimental import pallas as pl
from jax.experimental.pallas import tpu as pltpu
from jax.experimental.pallas import tpu_sc as plsc

F = 26
V = 1000
TOT = F * V
D = 16
B = 4096
NP0 = F * (F - 1) // 2
NPP = 328
RD = NPP * D // 128
TW = 27 * D
_PI = [i for i in range(F - 1) for j in range(i + 1, F)]
_PJ = [j for i in range(F - 1) for j in range(i + 1, F)]

NC, NS = 2, 16
NW = NC * NS
SPW = B // NW
SPC = 4
GR = SPC * F
NCH = SPW // SPC
KOUT = NCH // 2
WR = SPC * RD


def _sc_gather(tpw, idx):
    @functools.partial(
        pl.kernel,
        out_type=(
            jax.ShapeDtypeStruct((B * RD, 128), jnp.float32),
            jax.ShapeDtypeStruct((B, 128), jnp.float32),
        ),
        mesh=plsc.VectorSubcoreMesh(core_axis_name="c", subcore_axis_name="s",
                                    num_cores=NC, num_subcores=NS),
        compiler_params=pltpu.CompilerParams(use_tc_tiling_on_sc=False),
        scratch_types=[
            pltpu.VMEM((GR, TW), jnp.float32),
            pltpu.VMEM((GR, TW), jnp.float32),
            pltpu.VMEM((GR,), jnp.int32),
            pltpu.VMEM((GR,), jnp.int32),
            pltpu.VMEM((WR, 128), jnp.float32),
            pltpu.VMEM((SPC, 128), jnp.float32),
            pltpu.SemaphoreType.DMA,
            pltpu.SemaphoreType.DMA,
        ],
    )
    def body(tpw_hbm, idx_hbm, ix_hbm, plin_hbm,
             ga, gb, ia, ib, w4, linv, sga, sgb):
        wid = lax.axis_index("s") * NC + lax.axis_index("c")
        ibase = wid * (NCH * GR)
        zero16 = jnp.zeros((16,), jnp.float32)

        def fire(gv, iv, sem, gidx):
            base = jnp.minimum(ibase + gidx * GR, B * F - GR)
            pltpu.sync_copy(idx_hbm.at[pl.ds(base, GR)], iv)
            pltpu.async_copy(tpw_hbm.at[iv], gv, sem)

        def gwait(gv, sem):
            pltpu.make_async_copy(tpw_hbm.at[pl.ds(0, GR)], gv, sem).wait()

        def process(gv, ci):
            def persample(s, _):
                rb = s * F
                wrow = s * RD
                for p in range(NP0):
                    v = (gv[rb + _PI[p], pl.ds(D * _PJ[p], D)]
                         * gv[rb + _PJ[p], pl.ds(D * _PI[p], D)])
                    w4[wrow + p // 8, pl.ds(D * (p % 8), D)] = v
                for p in range(NP0, NPP):
                    w4[wrow + p // 8, pl.ds(D * (p % 8), D)] = zero16
                acc = gv[rb, pl.ds(26 * D, D)]
                for i in range(1, F):
                    acc = acc + gv[rb + i, pl.ds(26 * D, D)]
                linv[s, pl.ds(0, D)] = acc
                for z in range(1, 8):
                    linv[s, pl.ds(z * D, D)] = zero16
                return _

            lax.fori_loop(0, SPC, persample, 0)
            pltpu.sync_copy(
                w4, ix_hbm.at[pl.ds(wid * (SPW * RD) + ci * WR, WR)])
            pltpu.sync_copy(
                linv, plin_hbm.at[pl.ds(wid * SPW + ci * SPC, SPC)])

        fire(ga, ia, sga, 0)

        def outer(k, _):
            fire(gb, ib, sgb, 2 * k + 1)
            gwait(ga, sga)
            process(ga, 2 * k)
            fire(ga, ia, sga, 2 * k + 2)
            gwait(gb, sgb)
            process(gb, 2 * k + 1)
            return _

        lax.fori_loop(0, KOUT, outer, 0)
        gwait(ga, sga)

    return body(tpw, idx)


def _tc_mlp(ixd, plind, gsel, ones1, w1b, b1b, w2b4, b2b4, w3b2, b3b2):
    BB = 128
    NB = B // BB

    def body(ix_ref, plin_ref, g_ref, o1_ref,
             w1_ref, b1_ref, w2_ref, b2_ref, w3_ref, b3_ref,
             s1_ref, s2_ref):
        ixm = ix_ref[...]
        rs = jnp.dot(ixm, o1_ref[...],
                     preferred_element_type=jnp.float32)
        ffm = jnp.dot(g_ref[...], rs.astype(jnp.bfloat16),
                      preferred_element_type=jnp.float32)
        lin = jnp.dot(plin_ref[...], o1_ref[...],
                      preferred_element_type=jnp.float32)
        xmat = ixm.astype(jnp.bfloat16)
        h1 = jax.nn.relu(
            jnp.dot(xmat, w1_ref[...], preferred_element_type=jnp.float32)
            + b1_ref[...])
        ft = None
        for s in range(4):
            h1s = h1[:, 128 * s:128 * (s + 1)].astype(jnp.bfloat16)
            h2s = jax.nn.relu(
                jnp.dot(h1s, w2_ref[...],
                        preferred_element_type=jnp.float32)
                + b2_ref[...]).astype(jnp.bfloat16)
            fis = (jnp.dot(h2s, w3_ref[...],
                           preferred_element_type=jnp.float32)
                   + b3_ref[...])
            ft = fis if ft is None else ft + fis
        fr = ft[:, 0:1] + ft[:, 1:2]
        fisum = jnp.dot(g_ref[...], fr.astype(jnp.bfloat16),
                        preferred_element_type=jnp.float32)
        s1_ref[...] = lin + ffm
        s2_ref[...] = fisum

    return pl.pallas_call(
        body,
        grid=(NB,),
        in_specs=[
            pl.BlockSpec((BB * RD, 128), lambda i: (i, 0)),
            pl.BlockSpec((BB, 128), lambda i: (i, 0)),
            pl.BlockSpec((BB, BB * RD), lambda i: (0, 0)),
            pl.BlockSpec((128, 1), lambda i: (0, 0)),
            pl.BlockSpec((128, 512), lambda i: (0, 0)),
            pl.BlockSpec((1, 512), lambda i: (0, 0)),
            pl.BlockSpec((128, 64), lambda i: (0, 0)),
            pl.BlockSpec((1, 64), lambda i: (0, 0)),
            pl.BlockSpec((64, 2), lambda i: (0, 0)),
            pl.BlockSpec((1, 2), lambda i: (0, 0)),
        ],
        out_specs=[
            pl.BlockSpec((BB, 1), lambda i: (i, 0)),
            pl.BlockSpec((BB, 1), lambda i: (i, 0)),
        ],
        out_shape=[
            jax.ShapeDtypeStruct((B, 1), jnp.float32),
            jax.ShapeDtypeStruct((B, 1), jnp.float32),
        ],
    )(ixd, plind, gsel, ones1, w1b, b1b, w2b4, b2b4, w3b2, b3b2)


def _tc_outer(s1, s2t):
    RB = 256

    def body(s1_ref, s2_ref, out_ref):
        out_ref[...] = jax.nn.sigmoid(s1_ref[...] + s2_ref[...])

    return pl.pallas_call(
        body,
        grid=(B // RB,),
        in_specs=[
            pl.BlockSpec((RB, 1), lambda i: (i, 0)),
            pl.BlockSpec((1, B), lambda i: (0, 0)),
        ],
        out_specs=pl.BlockSpec((RB, B), lambda i: (i, 0)),
        out_shape=jax.ShapeDtypeStruct((B, B), jnp.float32),
    )(s1, s2t)


def kernel(x, lin_w, lin_b, tables, w1, b1, w2, b2, w3, b3):
    x = x.astype(jnp.int32)
    offs = (jnp.arange(F, dtype=jnp.int32) * V)[None, :]
    xo = x + offs
    idx = xo.reshape(-1)
    tpw = jnp.concatenate([
        tables.transpose(1, 0, 2).reshape(TOT, F * D),
        lin_w.reshape(TOT, 1),
        jnp.zeros((TOT, D - 1), jnp.float32),
    ], axis=1)

    ixd, plin = _sc_gather(tpw, idx)

    BB = 128
    gs = np.zeros((BB, BB * RD), np.float32)
    gs[np.arange(BB * RD) // RD, np.arange(BB * RD)] = 1.0
    gsel = jnp.asarray(gs)
    ones1 = jnp.ones((128, 1), jnp.float32)

    eye8 = jnp.eye(8, dtype=jnp.bfloat16)
    eye2 = jnp.eye(2, dtype=jnp.bfloat16)
    w1b = jnp.kron(eye8, w1.T.astype(jnp.bfloat16))
    w2b4 = jnp.kron(eye2, w2.T.astype(jnp.bfloat16))
    w3b2 = jnp.kron(eye2, w3.T.astype(jnp.bfloat16))
    b1b = jnp.tile(b1, 8)[None, :]
    b2b4 = jnp.tile(b2, 2)[None, :]
    b3b2 = jnp.tile(b3, 2)[None, :]

    s1, s2 = _tc_mlp(ixd, plin, gsel, ones1, w1b, b1b, w2b4, b2b4,
                     w3b2, b3b2)

    s1 = s1 + lin_b[0]
    cpad = (w3 @ jax.nn.relu(w2 @ jax.nn.relu(b1) + b2) + b3)[0]
    s2 = s2 - (NPP - NP0) * cpad

    return _tc_outer(s1, s2.reshape(1, B))


# speedup vs baseline: 1.1064x; 1.1064x over previous
import functools

import jax
import jax.numpy as jnp
import numpy as np
from jax import lax
from jax.experimental import pallas as pl
from jax.experimental.pallas import tpu as pltpu
from jax.experimental.pallas import tpu_sc as plsc

F = 26
V = 1000
TOT = F * V
D = 16
B = 4096
NP0 = F * (F - 1) // 2
NPP = 328
RD = NPP * D // 128
FP = 32
LD = FP * D // 128
ZROW = TOT
_PI = np.array([i for i in range(F - 1) for j in range(i + 1, F)], dtype=np.int32)
_PJ = np.array([j for i in range(F - 1) for j in range(i + 1, F)], dtype=np.int32)

NC, NS = 2, 16
NW = NC * NS
HROWS = B * NPP
HRPW = HROWS // NW
CH = 1024
NCHUNK = HRPW // CH
CW = CH * D // 128
LRPW = B * FP // NW
LNCH = LRPW // CH


def _sc_gather(tflat, linpad, idxp, idxl):
    @functools.partial(
        pl.kernel,
        out_type=(
            jax.ShapeDtypeStruct((HROWS * D // 128, 128), jnp.float32),
            jax.ShapeDtypeStruct((B * FP * D // 128, 128), jnp.float32),
        ),
        mesh=plsc.VectorSubcoreMesh(core_axis_name="c", subcore_axis_name="s",
                                    num_cores=NC, num_subcores=NS),
        compiler_params=pltpu.CompilerParams(use_tc_tiling_on_sc=False),
        scratch_types=[
            pltpu.VMEM((CH,), jnp.int32),
            pltpu.VMEM((CH,), jnp.int32),
            pltpu.VMEM((CH,), jnp.int32),
            pltpu.VMEM((CH,), jnp.int32),
            pltpu.VMEM((CH, D), jnp.float32),
            pltpu.VMEM((CH, D), jnp.float32),
            pltpu.VMEM((CH, D), jnp.float32),
            pltpu.VMEM((CH, D), jnp.float32),
            pltpu.VMEM((CW, 128), jnp.float32),
            pltpu.VMEM((CW, 128), jnp.float32),
            pltpu.SemaphoreType.DMA,
            pltpu.SemaphoreType.DMA,
            pltpu.SemaphoreType.DMA,
            pltpu.SemaphoreType.DMA,
        ],
    )
    def body(tflat_hbm, linpad_hbm, idxp_hbm, idxl_hbm, ix_hbm, plin_hbm,
             idx1a, idx1b, idx2a, idx2b, r1a, r2a, r1b, r2b, wa, wb,
             sga, sgb, swa, swb):
        wid = lax.axis_index("s") * NC + lax.axis_index("c")
        base0 = wid * HRPW
        idx1_v = (idx1a, idx1b)
        idx2_v = (idx2a, idx2b)
        r1_v = (r1a, r1b)
        r2_v = (r2a, r2b)
        w_v = (wa, wb)
        sg = (sga, sgb)
        sw = (swa, swb)

        def fire(c):
            p = c % 2
            base = base0 + c * CH
            pltpu.sync_copy(idxp_hbm.at[pl.ds(base, CH)], idx1_v[p])
            g1 = pltpu.async_copy(tflat_hbm.at[idx1_v[p]], r1_v[p], sg[p])
            pltpu.sync_copy(idxp_hbm.at[pl.ds(HROWS + base, CH)], idx2_v[p])
            g2 = pltpu.async_copy(tflat_hbm.at[idx2_v[p]], r2_v[p], sg[p])
            return g1, g2

        def mulpack(r1, r2, w):
            def step(r, _):
                for k in range(8):
                    w[r, pl.ds(16 * k, 16)] = (
                        r1[8 * r + k, :] * r2[8 * r + k, :])
                return 0
            lax.fori_loop(0, CW, step, 0)

        pend = fire(0)
        wpend = [None, None]
        for c in range(NCHUNK):
            p = c % 2
            nxt = fire(c + 1) if c + 1 < NCHUNK else None
            pend[0].wait()
            pend[1].wait()
            pend = nxt
            if wpend[p] is not None:
                wpend[p].wait()
            mulpack(r1_v[p], r2_v[p], w_v[p])
            wpend[p] = pltpu.async_copy(
                w_v[p],
                ix_hbm.at[pl.ds((base0 + c * CH) * D // 128, CW)], sw[p])
        for p in range(2):
            if wpend[p] is not None:
                wpend[p].wait()

        lbase0 = wid * LRPW
        for c in range(LNCH):
            base = lbase0 + c * CH
            pltpu.sync_copy(idxl_hbm.at[pl.ds(base, CH)], idx1a)
            pltpu.async_copy(linpad_hbm.at[idx1a], r1a, sga).wait()

            def lstep(r, _):
                for k in range(8):
                    wa[r, pl.ds(16 * k, 16)] = r1a[8 * r + k, :]
                return 0
            lax.fori_loop(0, CW, lstep, 0)
            pltpu.sync_copy(wa, plin_hbm.at[pl.ds(base * D // 128, CW)])

    return body(tflat, linpad, idxp, idxl)


def _tc_mlp(ixd, plind, maskm, gsel, gsel2, ones1,
            w1b, b1b, w2b4, b2b4, w3b2, b3b2):
    BB = 128
    NB = B // BB

    def body(ix_ref, plin_ref, mask_ref, g_ref, g2_ref, o1_ref,
             w1_ref, b1_ref, w2_ref, b2_ref, w3_ref, b3_ref,
             s1_ref, s2_ref):
        ixm = ix_ref[...] * mask_ref[...]
        rs = jnp.dot(ixm, o1_ref[...],
                     preferred_element_type=jnp.float32)
        ffm = jnp.dot(g_ref[...], rs.astype(jnp.bfloat16),
                      preferred_element_type=jnp.float32)
        rs2 = jnp.dot(plin_ref[...], o1_ref[...],
                      preferred_element_type=jnp.float32)
        lin = jnp.dot(g2_ref[...], rs2.astype(jnp.bfloat16),
                      preferred_element_type=jnp.float32)
        xmat = ixm.astype(jnp.bfloat16)
        h1 = jax.nn.relu(
            jnp.dot(xmat, w1_ref[...], preferred_element_type=jnp.float32)
            + b1_ref[...])
        ft = None
        for s in range(4):
            h1s = h1[:, 128 * s:128 * (s + 1)].astype(jnp.bfloat16)
            h2s = jax.nn.relu(
                jnp.dot(h1s, w2_ref[...],
                        preferred_element_type=jnp.float32)
                + b2_ref[...]).astype(jnp.bfloat16)
            fis = (jnp.dot(h2s, w3_ref[...],
                           preferred_element_type=jnp.float32)
                   + b3_ref[...])
            ft = fis if ft is None else ft + fis
        fr = ft[:, 0:1] + ft[:, 1:2]
        fisum = jnp.dot(g_ref[...], fr.astype(jnp.bfloat16),
                        preferred_element_type=jnp.float32)
        s1_ref[...] = lin + ffm
        s2_ref[...] = fisum

    return pl.pallas_call(
        body,
        grid=(NB,),
        in_specs=[
            pl.BlockSpec((BB * RD, 128), lambda i: (i, 0)),
            pl.BlockSpec((BB * LD, 128), lambda i: (i, 0)),
            pl.BlockSpec((BB * RD, 128), lambda i: (0, 0)),
            pl.BlockSpec((BB, BB * RD), lambda i: (0, 0)),
            pl.BlockSpec((BB, BB * LD), lambda i: (0, 0)),
            pl.BlockSpec((128, 1), lambda i: (0, 0)),
            pl.BlockSpec((128, 512), lambda i: (0, 0)),
            pl.BlockSpec((1, 512), lambda i: (0, 0)),
            pl.BlockSpec((128, 64), lambda i: (0, 0)),
            pl.BlockSpec((1, 64), lambda i: (0, 0)),
            pl.BlockSpec((64, 2), lambda i: (0, 0)),
            pl.BlockSpec((1, 2), lambda i: (0, 0)),
        ],
        out_specs=[
            pl.BlockSpec((BB, 1), lambda i: (i, 0)),
            pl.BlockSpec((BB, 1), lambda i: (i, 0)),
        ],
        out_shape=[
            jax.ShapeDtypeStruct((B, 1), jnp.float32),
            jax.ShapeDtypeStruct((B, 1), jnp.float32),
        ],
    )(ixd, plind, maskm, gsel, gsel2, ones1,
      w1b, b1b, w2b4, b2b4, w3b2, b3b2)


def _tc_outer(s1, s2t):
    RB = 256

    def body(s1_ref, s2_ref, out_ref):
        out_ref[...] = jax.nn.sigmoid(s1_ref[...] + s2_ref[...])

    return pl.pallas_call(
        body,
        grid=(B // RB,),
        in_specs=[
            pl.BlockSpec((RB, 1), lambda i: (i, 0)),
            pl.BlockSpec((1, B), lambda i: (0, 0)),
        ],
        out_specs=pl.BlockSpec((RB, B), lambda i: (i, 0)),
        out_shape=jax.ShapeDtypeStruct((B, B), jnp.float32),
    )(s1, s2t)


def kernel(x, lin_w, lin_b, tables, w1, b1, w2, b2, w3, b3):
    x = x.astype(jnp.int32)
    offs = (jnp.arange(F, dtype=jnp.int32) * V)[None, :]
    xo = x + offs
    c1 = np.concatenate([_PI, np.zeros(NPP - NP0, np.int32)])
    a1 = np.concatenate([_PJ * TOT, np.zeros(NPP - NP0, np.int32)])
    c2 = np.concatenate([_PJ, np.zeros(NPP - NP0, np.int32)])
    a2 = np.concatenate([_PI * TOT, np.zeros(NPP - NP0, np.int32)])
    idx1 = jnp.take(xo, jnp.asarray(c1), axis=1) + jnp.asarray(a1)[None, :]
    idx2 = jnp.take(xo, jnp.asarray(c2), axis=1) + jnp.asarray(a2)[None, :]
    idxp = jnp.concatenate([idx1, idx2], axis=0).reshape(-1)
    idxl = jnp.concatenate(
        [xo, jnp.full((B, FP - F), ZROW, jnp.int32)], axis=1).reshape(-1)
    linpad = jnp.zeros((TOT + 8, D), jnp.float32).at[:TOT, 0].set(
        lin_w.reshape(TOT))

    ixd, plin = _sc_gather(tables.reshape(F * TOT, D), linpad, idxp, idxl)

    BB = 128
    mrow = np.ones((RD, 128), np.float32)
    mrow[RD - 1, 128 - (NPP - NP0) * D:] = 0.0
    maskm = jnp.asarray(np.tile(mrow, (BB, 1)))
    gs = np.zeros((BB, BB * RD), np.float32)
    gs[np.arange(BB * RD) // RD, np.arange(BB * RD)] = 1.0
    gsel = jnp.asarray(gs)
    gs2 = np.zeros((BB, BB * LD), np.float32)
    gs2[np.arange(BB * LD) // LD, np.arange(BB * LD)] = 1.0
    gsel2 = jnp.asarray(gs2)
    ones1 = jnp.ones((128, 1), jnp.float32)

    eye8 = jnp.eye(8, dtype=jnp.bfloat16)
    eye2 = jnp.eye(2, dtype=jnp.bfloat16)
    w1b = jnp.kron(eye8, w1.T.astype(jnp.bfloat16))
    w2b4 = jnp.kron(eye2, w2.T.astype(jnp.bfloat16))
    w3b2 = jnp.kron(eye2, w3.T.astype(jnp.bfloat16))
    b1b = jnp.tile(b1, 8)[None, :]
    b2b4 = jnp.tile(b2, 2)[None, :]
    b3b2 = jnp.tile(b3, 2)[None, :]

    s1, s2 = _tc_mlp(ixd, plin, maskm, gsel, gsel2, ones1,
                     w1b, b1b, w2b4, b2b4, w3b2, b3b2)

    s1 = s1 + lin_b[0]
    cpad = (w3 @ jax.nn.relu(w2 @ jax.nn.relu(b1) + b2) + b3)[0]
    s2 = s2 - (NPP - NP0) * cpad

    return _tc_outer(s1, s2.reshape(1, B))


# speedup vs baseline: 1.1515x; 1.0408x over previous
import functools

import jax
import jax.numpy as jnp
import numpy as np
from jax import lax
from jax.experimental import pallas as pl
from jax.experimental.pallas import tpu as pltpu
from jax.experimental.pallas import tpu_sc as plsc

F = 26
V = 1000
TOT = F * V
D = 16
B = 4096
NP0 = F * (F - 1) // 2
NPP = 328
RD = NPP * D // 128
FP = 32
LD = FP * D // 128
ZROW = TOT
_PI = np.array([i for i in range(F - 1) for j in range(i + 1, F)], dtype=np.int32)
_PJ = np.array([j for i in range(F - 1) for j in range(i + 1, F)], dtype=np.int32)

NC, NS = 2, 16
NW = NC * NS
NSPLIT = 2
NB_ = B // NSPLIT


def _sc_gather(tflat, linpad, idxp, idxl):
    HROWS = NB_ * NPP
    HRPW = HROWS // NW
    CH = 512
    NCHUNK = HRPW // CH
    CW = CH * D // 128
    LRPW = NB_ * FP // NW
    LNCH = LRPW // CH

    @functools.partial(
        pl.kernel,
        out_type=(
            jax.ShapeDtypeStruct((HROWS * D // 128, 128), jnp.float32),
            jax.ShapeDtypeStruct((NB_ * FP * D // 128, 128), jnp.float32),
        ),
        mesh=plsc.VectorSubcoreMesh(core_axis_name="c", subcore_axis_name="s",
                                    num_cores=NC, num_subcores=NS),
        compiler_params=pltpu.CompilerParams(use_tc_tiling_on_sc=False),
        scratch_types=[
            pltpu.VMEM((CH,), jnp.int32),
            pltpu.VMEM((CH,), jnp.int32),
            pltpu.VMEM((CH,), jnp.int32),
            pltpu.VMEM((CH,), jnp.int32),
            pltpu.VMEM((CH, D), jnp.float32),
            pltpu.VMEM((CH, D), jnp.float32),
            pltpu.VMEM((CH, D), jnp.float32),
            pltpu.VMEM((CH, D), jnp.float32),
            pltpu.VMEM((CW, 128), jnp.float32),
            pltpu.VMEM((CW, 128), jnp.float32),
            pltpu.SemaphoreType.DMA,
            pltpu.SemaphoreType.DMA,
            pltpu.SemaphoreType.DMA,
            pltpu.SemaphoreType.DMA,
        ],
    )
    def body(tflat_hbm, linpad_hbm, idxp_hbm, idxl_hbm, ix_hbm, plin_hbm,
             idx1a, idx1b, idx2a, idx2b, r1a, r2a, r1b, r2b, wa, wb,
             sga, sgb, swa, swb):
        wid = lax.axis_index("s") * NC + lax.axis_index("c")
        base0 = wid * HRPW
        idx1_v = (idx1a, idx1b)
        idx2_v = (idx2a, idx2b)
        r1_v = (r1a, r1b)
        r2_v = (r2a, r2b)
        w_v = (wa, wb)
        sg = (sga, sgb)
        sw = (swa, swb)

        def fire(c):
            p = c % 2
            base = base0 + c * CH
            pltpu.sync_copy(idxp_hbm.at[pl.ds(base, CH)], idx1_v[p])
            g1 = pltpu.async_copy(tflat_hbm.at[idx1_v[p]], r1_v[p], sg[p])
            pltpu.sync_copy(idxp_hbm.at[pl.ds(HROWS + base, CH)], idx2_v[p])
            g2 = pltpu.async_copy(tflat_hbm.at[idx2_v[p]], r2_v[p], sg[p])
            return g1, g2

        def mulpack(r1, r2, w):
            def step(r, _):
                for k in range(8):
                    w[r, pl.ds(16 * k, 16)] = (
                        r1[8 * r + k, :] * r2[8 * r + k, :])
                return 0
            lax.fori_loop(0, CW, step, 0)

        pend = fire(0)
        wpend = [None, None]
        for c in range(NCHUNK):
            p = c % 2
            nxt = fire(c + 1) if c + 1 < NCHUNK else None
            pend[0].wait()
            pend[1].wait()
            pend = nxt
            if wpend[p] is not None:
                wpend[p].wait()
            mulpack(r1_v[p], r2_v[p], w_v[p])
            wpend[p] = pltpu.async_copy(
                w_v[p],
                ix_hbm.at[pl.ds((base0 + c * CH) * D // 128, CW)], sw[p])
        for p in range(2):
            if wpend[p] is not None:
                wpend[p].wait()

        lbase0 = wid * LRPW
        for c in range(LNCH):
            base = lbase0 + c * CH
            pltpu.sync_copy(idxl_hbm.at[pl.ds(base, CH)], idx1a)
            pltpu.async_copy(linpad_hbm.at[idx1a], r1a, sga).wait()

            def lstep(r, _):
                for k in range(8):
                    wa[r, pl.ds(16 * k, 16)] = r1a[8 * r + k, :]
                return 0
            lax.fori_loop(0, CW, lstep, 0)
            pltpu.sync_copy(wa, plin_hbm.at[pl.ds(base * D // 128, CW)])

    return body(tflat, linpad, idxp, idxl)


def _tc_mlp(ixd, plind, maskm, gsel, gsel2, ones1,
            w1b, b1b, w2b4, b2b4, w3b2, b3b2):
    BB = 128
    NB = NB_ // BB

    def body(ix_ref, plin_ref, mask_ref, g_ref, g2_ref, o1_ref,
             w1_ref, b1_ref, w2_ref, b2_ref, w3_ref, b3_ref,
             s1_ref, s2_ref):
        ixm = ix_ref[...] * mask_ref[...]
        rs = jnp.dot(ixm, o1_ref[...],
                     preferred_element_type=jnp.float32)
        ffm = jnp.dot(g_ref[...], rs.astype(jnp.bfloat16),
                      preferred_element_type=jnp.float32)
        rs2 = jnp.dot(plin_ref[...], o1_ref[...],
                      preferred_element_type=jnp.float32)
        lin = jnp.dot(g2_ref[...], rs2.astype(jnp.bfloat16),
                      preferred_element_type=jnp.float32)
        xmat = ixm.astype(jnp.bfloat16)
        h1 = jax.nn.relu(
            jnp.dot(xmat, w1_ref[...], preferred_element_type=jnp.float32)
            + b1_ref[...])
        ft = None
        for s in range(4):
            h1s = h1[:, 128 * s:128 * (s + 1)].astype(jnp.bfloat16)
            h2s = jax.nn.relu(
                jnp.dot(h1s, w2_ref[...],
                        preferred_element_type=jnp.float32)
                + b2_ref[...]).astype(jnp.bfloat16)
            fis = (jnp.dot(h2s, w3_ref[...],
                           preferred_element_type=jnp.float32)
                   + b3_ref[...])
            ft = fis if ft is None else ft + fis
        fr = ft[:, 0:1] + ft[:, 1:2]
        fisum = jnp.dot(g_ref[...], fr.astype(jnp.bfloat16),
                        preferred_element_type=jnp.float32)
        s1_ref[...] = lin + ffm
        s2_ref[...] = fisum

    return pl.pallas_call(
        body,
        grid=(NB,),
        in_specs=[
            pl.BlockSpec((BB * RD, 128), lambda i: (i, 0)),
            pl.BlockSpec((BB * LD, 128), lambda i: (i, 0)),
            pl.BlockSpec((BB * RD, 128), lambda i: (0, 0)),
            pl.BlockSpec((BB, BB * RD), lambda i: (0, 0)),
            pl.BlockSpec((BB, BB * LD), lambda i: (0, 0)),
            pl.BlockSpec((128, 1), lambda i: (0, 0)),
            pl.BlockSpec((128, 512), lambda i: (0, 0)),
            pl.BlockSpec((1, 512), lambda i: (0, 0)),
            pl.BlockSpec((128, 64), lambda i: (0, 0)),
            pl.BlockSpec((1, 64), lambda i: (0, 0)),
            pl.BlockSpec((64, 2), lambda i: (0, 0)),
            pl.BlockSpec((1, 2), lambda i: (0, 0)),
        ],
        out_specs=[
            pl.BlockSpec((BB, 1), lambda i: (i, 0)),
            pl.BlockSpec((BB, 1), lambda i: (i, 0)),
        ],
        out_shape=[
            jax.ShapeDtypeStruct((NB_, 1), jnp.float32),
            jax.ShapeDtypeStruct((NB_, 1), jnp.float32),
        ],
    )(ixd, plind, maskm, gsel, gsel2, ones1,
      w1b, b1b, w2b4, b2b4, w3b2, b3b2)


def _tc_outer(s1, s2t):
    RB = 256

    def body(s1_ref, s2_ref, out_ref):
        out_ref[...] = jax.nn.sigmoid(s1_ref[...] + s2_ref[...])

    return pl.pallas_call(
        body,
        grid=(B // RB,),
        in_specs=[
            pl.BlockSpec((RB, 1), lambda i: (i, 0)),
            pl.BlockSpec((1, B), lambda i: (0, 0)),
        ],
        out_specs=pl.BlockSpec((RB, B), lambda i: (i, 0)),
        out_shape=jax.ShapeDtypeStruct((B, B), jnp.float32),
    )(s1, s2t)


def kernel(x, lin_w, lin_b, tables, w1, b1, w2, b2, w3, b3):
    x = x.astype(jnp.int32)
    offs = (jnp.arange(F, dtype=jnp.int32) * V)[None, :]
    xo = x + offs
    c1 = np.concatenate([_PI, np.zeros(NPP - NP0, np.int32)])
    a1 = np.concatenate([_PJ * TOT, np.zeros(NPP - NP0, np.int32)])
    c2 = np.concatenate([_PJ, np.zeros(NPP - NP0, np.int32)])
    a2 = np.concatenate([_PI * TOT, np.zeros(NPP - NP0, np.int32)])
    idx1 = jnp.take(xo, jnp.asarray(c1), axis=1) + jnp.asarray(a1)[None, :]
    idx2 = jnp.take(xo, jnp.asarray(c2), axis=1) + jnp.asarray(a2)[None, :]
    idxl = jnp.concatenate(
        [xo, jnp.full((B, FP - F), ZROW, jnp.int32)], axis=1)
    linpad = jnp.zeros((TOT + 8, D), jnp.float32).at[:TOT, 0].set(
        lin_w.reshape(TOT))
    tflat = tables.reshape(F * TOT, D)

    halves = []
    for h in range(NSPLIT):
        lo = h * NB_
        idxp_h = jnp.concatenate(
            [lax.slice_in_dim(idx1, lo, lo + NB_, axis=0),
             lax.slice_in_dim(idx2, lo, lo + NB_, axis=0)],
            axis=0).reshape(-1)
        idxl_h = lax.slice_in_dim(idxl, lo, lo + NB_, axis=0).reshape(-1)
        halves.append(_sc_gather(tflat, linpad, idxp_h, idxl_h))

    BB = 128
    mrow = np.ones((RD, 128), np.float32)
    mrow[RD - 1, 128 - (NPP - NP0) * D:] = 0.0
    maskm = jnp.asarray(np.tile(mrow, (BB, 1)))
    gs = np.zeros((BB, BB * RD), np.float32)
    gs[np.arange(BB * RD) // RD, np.arange(BB * RD)] = 1.0
    gsel = jnp.asarray(gs)
    gs2 = np.zeros((BB, BB * LD), np.float32)
    gs2[np.arange(BB * LD) // LD, np.arange(BB * LD)] = 1.0
    gsel2 = jnp.asarray(gs2)
    ones1 = jnp.ones((128, 1), jnp.float32)

    eye8 = jnp.eye(8, dtype=jnp.bfloat16)
    eye2 = jnp.eye(2, dtype=jnp.bfloat16)
    w1b = jnp.kron(eye8, w1.T.astype(jnp.bfloat16))
    w2b4 = jnp.kron(eye2, w2.T.astype(jnp.bfloat16))
    w3b2 = jnp.kron(eye2, w3.T.astype(jnp.bfloat16))
    b1b = jnp.tile(b1, 8)[None, :]
    b2b4 = jnp.tile(b2, 2)[None, :]
    b3b2 = jnp.tile(b3, 2)[None, :]

    parts = [_tc_mlp(ixd, plin, maskm, gsel, gsel2, ones1,
                     w1b, b1b, w2b4, b2b4, w3b2, b3b2)
             for ixd, plin in halves]
    s1 = jnp.concatenate([p[0] for p in parts], axis=0)
    s2 = jnp.concatenate([p[1] for p in parts], axis=0)

    s1 = s1 + lin_b[0]
    cpad = (w3 @ jax.nn.relu(w2 @ jax.nn.relu(b1) + b2) + b3)[0]
    s2 = s2 - (NPP - NP0) * cpad

    return _tc_outer(s1, s2.reshape(1, B))
